# Initial kernel scaffold; baseline (speedup 1.0000x reference)
import jax
import jax.numpy as jnp
from jax.experimental import pallas as pl


def kernel(x, edge_index, edge_attr, schedule, y, Wn, bn, We, be, Wm, bm, Wu, bu, W1, b1, W2, b2, W3, b3):
    raise NotImplementedError("write your pallas kernel here")


# speedup vs baseline: 2.6319x; 2.6319x over previous
import functools

import jax
import jax.numpy as jnp
from jax import lax
from jax.experimental import pallas as pl
from jax.experimental.pallas import tpu as pltpu
from jax.experimental.pallas import tpu_sc as plsc

NG = 10
NPG = 1000
NN = NG * NPG
NE = 320000
DN = 128
DE = 16
DM = 128
NL = 3
SPG = 500
NP = NG * SPG * 2

NW = 32
NNPAD = 10240
EPW = NE // NW
EC = 80
NCHUNK = EPW // EC
RPS = NNPAD // 16
NPPAD = NW * 320
GC = 80


def _sc_mesh():
    return plsc.VectorSubcoreMesh(core_axis_name="c", subcore_axis_name="s")


def _weights_body(we, be, wm, bm, ke_o, cb_o):
    for l in range(NL):
        ke_o[l] = jnp.dot(we[...], wm[l], preferred_element_type=jnp.float32)
        cb_o[l] = jnp.dot(be[...], wm[l], preferred_element_type=jnp.float32) + bm[l]


def _weights_call(We, be2, Wm, bm3):
    return pl.pallas_call(
        _weights_body,
        out_shape=(
            jax.ShapeDtypeStruct((NL, DE, DM), jnp.float32),
            jax.ShapeDtypeStruct((NL, 1, DM), jnp.float32),
        ),
    )(We, be2, Wm, bm3)


def _enc_body(xb, wn, bn, wm0, cb0, nf_o, p_o):
    nf = jnp.dot(xb[...], wn[...], preferred_element_type=jnp.float32) + bn[...]
    nf_o[...] = nf
    p_o[...] = jnp.dot(nf, wm0[...], preferred_element_type=jnp.float32) + cb0[...]


def _enc_call(x, Wn, bn2, Wm0, cb0):
    blk = 1000
    grid = NN // blk
    return pl.pallas_call(
        _enc_body,
        grid=(grid,),
        in_specs=[
            pl.BlockSpec((blk, DN), lambda i: (i, 0)),
            pl.BlockSpec((DN, DM), lambda i: (0, 0)),
            pl.BlockSpec((1, DM), lambda i: (0, 0)),
            pl.BlockSpec((DM, DM), lambda i: (0, 0)),
            pl.BlockSpec((1, DM), lambda i: (0, 0)),
        ],
        out_specs=(
            pl.BlockSpec((blk, DM), lambda i: (i, 0)),
            pl.BlockSpec((blk, DM), lambda i: (i, 0)),
        ),
        out_shape=(
            jax.ShapeDtypeStruct((NN, DM), jnp.float32),
            jax.ShapeDtypeStruct((NN, DM), jnp.float32),
        ),
    )(x, Wn, bn2, Wm0, cb0)


def _q_body(ab, ke, q_o):
    q_o[...] = jnp.dot(ab[...], ke[...], preferred_element_type=jnp.float32)


def _q_call(edge_attr, Ke_l):
    blk = 8000
    grid = NE // blk
    return pl.pallas_call(
        _q_body,
        grid=(grid,),
        in_specs=[
            pl.BlockSpec((blk, DE), lambda i: (i, 0)),
            pl.BlockSpec((DE, DM), lambda i: (0, 0)),
        ],
        out_specs=pl.BlockSpec((blk, DM), lambda i: (i, 0)),
        out_shape=jax.ShapeDtypeStruct((NE, DM), jnp.float32),
    )(edge_attr, Ke_l)


def _upd_body(nfb, ab, wu, bu, wmn, cbn, nf_o, p_o):
    aggr = ab[0] + ab[1]
    t = jnp.dot(nfb[...] + aggr, wu[...], preferred_element_type=jnp.float32) + bu[...]
    nf2 = nfb[...] + jnp.maximum(t, 0.0)
    nf_o[...] = nf2
    p_o[...] = jnp.dot(nf2, wmn[...], preferred_element_type=jnp.float32) + cbn[...]


def _upd_call(nf, aggr2, Wu_l, bu2, Wm_n, cb_n):
    blk = 1000
    grid = NN // blk
    return pl.pallas_call(
        _upd_body,
        grid=(grid,),
        in_specs=[
            pl.BlockSpec((blk, DM), lambda i: (i, 0)),
            pl.BlockSpec((2, blk, DM), lambda i: (0, i, 0)),
            pl.BlockSpec((DM, DM), lambda i: (0, 0)),
            pl.BlockSpec((1, DM), lambda i: (0, 0)),
            pl.BlockSpec((DM, DM), lambda i: (0, 0)),
            pl.BlockSpec((1, DM), lambda i: (0, 0)),
        ],
        out_specs=(
            pl.BlockSpec((blk, DM), lambda i: (i, 0)),
            pl.BlockSpec((blk, DM), lambda i: (i, 0)),
        ),
        out_shape=(
            jax.ShapeDtypeStruct((NN, DM), jnp.float32),
            jax.ShapeDtypeStruct((NN, DM), jnp.float32),
        ),
    )(nf, aggr2, Wu_l, bu2, Wm_n, cb_n)


def _mlp_body(h, yb, w1, b1, w2, b2, w3, b3, o):
    a = jnp.maximum(jnp.dot(h[...], w1[...], preferred_element_type=jnp.float32) + b1[...], 0.0)
    b = jnp.maximum(jnp.dot(a, w2[...], preferred_element_type=jnp.float32) + b2[...], 0.0)
    yh = jnp.dot(b, w3[...], preferred_element_type=jnp.float32) + b3[...]
    d = yh - yb[...]
    o[...] = (jnp.sum(d * d) / (NG * SPG)).reshape(1, 1)


def _mlp_call(h, y, W1, b12, W2, b22, W3, b32):
    return pl.pallas_call(
        _mlp_body,
        out_shape=jax.ShapeDtypeStruct((1, 1), jnp.float32),
    )(h, y, W1, b12, W2, b22, W3, b32)


def _edge_body(p_hbm, q_hbm, src_hbm, dst_hbm, out_hbm,
               src_v, dst_v, msg_v, prw_v, aggr_sh, sem):
    c = lax.axis_index("c")
    s = lax.axis_index("s")

    zero16 = jnp.zeros((16,), jnp.float32)

    def zrow(r, carry):
        for j in range(DM // 16):
            msg_v[r, pl.ds(j * 16, 16)] = zero16
        return carry

    lax.fori_loop(0, EC, zrow, 0)
    for k in range(RPS // EC):
        pltpu.sync_copy(msg_v, aggr_sh.at[pl.ds(s * RPS + k * EC, EC)])
    plsc.subcore_barrier()

    base = (c * 16 + s) * EPW

    def chunk(i, carry):
        off = base + i * EC
        pltpu.sync_copy(src_hbm.at[pl.ds(off, EC)], src_v)
        pltpu.sync_copy(dst_hbm.at[pl.ds(off, EC)], dst_v)
        pltpu.sync_copy(q_hbm.at[pl.ds(off, EC)], msg_v)
        pltpu.async_copy(p_hbm.at[src_v], prw_v, sem).wait()

        def row(r, rc):
            for j in range(DM // 16):
                sl = pl.ds(j * 16, 16)
                msg_v[r, sl] = jnp.maximum(msg_v[r, sl] + prw_v[r, sl], 0.0)
            return rc

        lax.fori_loop(0, EC, row, 0)
        pltpu.sync_copy(msg_v, aggr_sh.at[dst_v], add=True)
        return carry

    lax.fori_loop(0, NCHUNK, chunk, 0)
    plsc.subcore_barrier()
    for k in range(RPS // EC):
        r0 = s * RPS + k * EC
        pltpu.sync_copy(aggr_sh.at[pl.ds(r0, EC)], out_hbm.at[c, pl.ds(r0, EC)])


@functools.partial(
    pl.kernel,
    out_type=jax.ShapeDtypeStruct((2, NNPAD, DM), jnp.float32),
    mesh=_sc_mesh(),
    scratch_types=[
        pltpu.VMEM((EC,), jnp.int32),
        pltpu.VMEM((EC,), jnp.int32),
        pltpu.VMEM((EC, DM), jnp.float32),
        pltpu.VMEM((EC, DM), jnp.float32),
        pltpu.VMEM_SHARED((NNPAD, DM), jnp.float32),
        pltpu.SemaphoreType.DMA,
    ],
)
def _edge_call(p_hbm, q_hbm, src_hbm, dst_hbm, out_hbm,
               src_v, dst_v, msg_v, prw_v, aggr_sh, sem):
    _edge_body(p_hbm, q_hbm, src_hbm, dst_hbm, out_hbm,
               src_v, dst_v, msg_v, prw_v, aggr_sh, sem)


@functools.partial(
    pl.kernel,
    out_type=jax.ShapeDtypeStruct((NPPAD, DM), jnp.float32),
    mesh=_sc_mesh(),
    scratch_types=[
        pltpu.VMEM((GC,), jnp.int32),
        pltpu.VMEM((GC, DM), jnp.float32),
        pltpu.SemaphoreType.DMA,
    ],
)
def _pair_gather(tbl_hbm, idx_hbm, out_hbm, idx_v, row_v, sem):
    c = lax.axis_index("c")
    s = lax.axis_index("s")
    base = (c * 16 + s) * (NPPAD // NW)

    def chunk(i, carry):
        off = base + i * GC
        pltpu.sync_copy(idx_hbm.at[pl.ds(off, GC)], idx_v)
        pltpu.async_copy(tbl_hbm.at[idx_v], row_v, sem).wait()
        pltpu.sync_copy(row_v, out_hbm.at[pl.ds(off, GC)])
        return carry

    lax.fori_loop(0, (NPPAD // NW) // GC, chunk, 0)


def kernel(x, edge_index, edge_attr, schedule, y,
           Wn, bn, We, be, Wm, bm, Wu, bu, W1, b1, W2, b2, W3, b3):
    src = edge_index[0]
    dst = edge_index[1]
    bn2 = bn.reshape(1, DM)
    be2 = be.reshape(1, DM)
    bm3 = bm.reshape(NL, 1, DM)

    Ke, cb = _weights_call(We, be2, Wm, bm3)
    nf, P = _enc_call(x, Wn, bn2, Wm[0], cb[0])
    for l in range(NL):
        Q = _q_call(edge_attr, Ke[l])
        aggr2 = _edge_call(P, Q, src, dst)
        nf, P = _upd_call(nf, aggr2, Wu[l], bu[l].reshape(1, DM),
                          Wm[(l + 1) % NL], cb[(l + 1) % NL])

    offs = (jnp.arange(NG, dtype=jnp.int32) * NPG)[:, None]
    flat = (schedule.reshape(NG, SPG * 2) + offs).reshape(-1)
    flat = jnp.concatenate(
        [flat, jnp.zeros((NPPAD - NP,), jnp.int32)])
    pairs = _pair_gather(nf, flat)
    h = pairs[:NP].reshape(NG * SPG, 2 * DM)

    loss = _mlp_call(h, y, W1, b1.reshape(1, DM), W2, b2.reshape(1, DM // 2),
                     W3, b3.reshape(1, 1))
    return loss.reshape(())
